# baseline (speedup 1.0000x reference)
import jax
import jax.numpy as jnp
from jax.experimental import pallas as pl
from jax.experimental.pallas import tpu as pltpu


def _vq_body(z_ref, e_ref, zq_ref, loss_ref,
             em2_ref, ecat_ref, esq_ref, sse_ref):
    k, c = e_ref.shape
    p = z_ref.shape[2]
    h, w = zq_ref.shape[2], zq_ref.shape[3]
    nsteps = pl.num_programs(0)

    @pl.when(pl.program_id(0) == 0)
    def _prep():
        e = e_ref[...]
        esq_ref[...] = jnp.sum(e * e, axis=1, keepdims=True)
        em2_ref[...] = -2.0 * e
        e0 = e.astype(jnp.bfloat16)
        r1 = e - e0.astype(jnp.float32)
        e1 = r1.astype(jnp.bfloat16)
        e2c = (r1 - e1.astype(jnp.float32)).astype(jnp.bfloat16)
        ecat_ref[:, 0:c] = e0
        ecat_ref[:, c:2 * c] = e1
        ecat_ref[:, 2 * c:3 * c] = e2c
        sse_ref[0, 0] = 0.0

    zc = z_ref[0]
    z2 = jnp.sum(zc * zc, axis=0, keepdims=True)
    mmn2 = jax.lax.dot_general(
        em2_ref[...], zc, (((1,), (0,)), ((), ())),
        preferred_element_type=jnp.float32)
    dist = (z2 + esq_ref[...]) + mmn2

    minv = jnp.min(dist, axis=0, keepdims=True)
    iota = jax.lax.broadcasted_iota(jnp.int32, (k, p), 0)
    idx = jnp.min(jnp.where(dist == minv, iota, k), axis=0, keepdims=True)

    onehot = (iota == idx).astype(jnp.bfloat16)
    parts = jax.lax.dot_general(
        ecat_ref[...], onehot, (((0,), (0,)), ((), ())),
        preferred_element_type=jnp.float32)
    zq = (parts[:c] + parts[c:2 * c]) + parts[2 * c:]

    d = zq - zc
    zq_ref[0] = (zc + d).reshape(c, h, w)
    sse_ref[0, 0] += jnp.sum(d * d)

    @pl.when(pl.program_id(0) == nsteps - 1)
    def _fin():
        m = sse_ref[0, 0] / (nsteps * c * p)
        loss_ref[0, 0] = m + 0.25 * m


def kernel(z, embedding):
    B, C, H, W = z.shape
    K = embedding.shape[0]
    P = H * W
    zv = z.reshape(B, C, P)

    zq, loss = pl.pallas_call(
        _vq_body,
        grid=(B,),
        in_specs=[
            pl.BlockSpec((1, C, P), lambda i: (i, 0, 0)),
            pl.BlockSpec((K, C), lambda i: (0, 0)),
        ],
        out_specs=[
            pl.BlockSpec((1, C, H, W), lambda i: (i, 0, 0, 0)),
            pl.BlockSpec(memory_space=pltpu.SMEM),
        ],
        out_shape=[
            jax.ShapeDtypeStruct((B, C, H, W), jnp.float32),
            jax.ShapeDtypeStruct((1, 1), jnp.float32),
        ],
        scratch_shapes=[
            pltpu.VMEM((K, C), jnp.float32),
            pltpu.VMEM((K, 3 * C), jnp.bfloat16),
            pltpu.VMEM((K, 1), jnp.float32),
            pltpu.SMEM((1, 1), jnp.float32),
        ],
    )(zv, embedding)

    return (zq, loss[0, 0])


# speedup vs baseline: 1.1517x; 1.1517x over previous
import jax
import jax.numpy as jnp
from jax.experimental import pallas as pl
from jax.experimental.pallas import tpu as pltpu


def _vq_body(z_ref, e_ref, zq_ref, loss_ref,
             em2_ref, ecat_ref, esq_ref, sse_ref):
    k, c = e_ref.shape
    p = z_ref.shape[2]
    nsteps = pl.num_programs(0)

    @pl.when(pl.program_id(0) == 0)
    def _prep():
        e = e_ref[...]
        esq_ref[...] = jnp.sum(e * e, axis=1, keepdims=True)
        em2_ref[...] = -2.0 * e
        e0 = e.astype(jnp.bfloat16)
        r1 = e - e0.astype(jnp.float32)
        e1 = r1.astype(jnp.bfloat16)
        e2c = (r1 - e1.astype(jnp.float32)).astype(jnp.bfloat16)
        ecat_ref[:, 0:c] = e0
        ecat_ref[:, c:2 * c] = e1
        ecat_ref[:, 2 * c:3 * c] = e2c
        sse_ref[0, 0] = 0.0

    zc = z_ref[0]
    z2 = jnp.sum(zc * zc, axis=0, keepdims=True)
    mmn2 = jax.lax.dot_general(
        em2_ref[...], zc, (((1,), (0,)), ((), ())),
        preferred_element_type=jnp.float32)
    dist = (z2 + esq_ref[...]) + mmn2

    minv = jnp.min(dist, axis=0, keepdims=True)
    iota = jax.lax.broadcasted_iota(jnp.int32, (k, p), 0)
    idx = jnp.min(jnp.where(dist == minv, iota, k), axis=0, keepdims=True)

    onehot = (iota == idx).astype(jnp.bfloat16)
    parts = jax.lax.dot_general(
        ecat_ref[...], onehot, (((0,), (0,)), ((), ())),
        preferred_element_type=jnp.float32)
    zq = (parts[:c] + parts[c:2 * c]) + parts[2 * c:]

    d = zq - zc
    zq_ref[0] = zc + d
    sse_ref[0, 0] += jnp.sum(d * d)

    @pl.when(pl.program_id(0) == nsteps - 1)
    def _fin():
        m = sse_ref[0, 0] / (nsteps * c * p)
        loss_ref[0, 0] = m + 0.25 * m


def kernel(z, embedding):
    B, C, H, W = z.shape
    K = embedding.shape[0]
    P = H * W
    zv = z.reshape(B, C, P)

    zq, loss = pl.pallas_call(
        _vq_body,
        grid=(B,),
        in_specs=[
            pl.BlockSpec((1, C, P), lambda i: (i, 0, 0)),
            pl.BlockSpec((K, C), lambda i: (0, 0)),
        ],
        out_specs=[
            pl.BlockSpec((1, C, P), lambda i: (i, 0, 0)),
            pl.BlockSpec(memory_space=pltpu.SMEM),
        ],
        out_shape=[
            jax.ShapeDtypeStruct((B, C, P), jnp.float32),
            jax.ShapeDtypeStruct((1, 1), jnp.float32),
        ],
        scratch_shapes=[
            pltpu.VMEM((K, C), jnp.float32),
            pltpu.VMEM((K, 3 * C), jnp.bfloat16),
            pltpu.VMEM((K, 1), jnp.float32),
            pltpu.SMEM((1, 1), jnp.float32),
        ],
    )(zv, embedding)

    return (zq.reshape(B, C, H, W), loss[0, 0])
